# baseline (speedup 1.0000x reference)
import jax
import jax.numpy as jnp
from jax import lax
from jax.experimental import pallas as pl
from jax.experimental.pallas import tpu as pltpu
from jax.experimental.pallas import tpu_sc as plsc

NUM_TOKENS = 8192
D_MODEL = 32
BATCH = 1024

CHUNK = 1024
G = NUM_TOKENS // CHUNK
DG = D_MODEL * G

_NC = 2
_NS = 16
_NW = _NC * _NS
_BPW = BATCH // _NW


def _hi(v):
    return v.astype(jnp.bfloat16).astype(jnp.float32)


def _argmin_body(x_ref, cb_ref, idx_ref):
    x = x_ref[...]
    zeros = jnp.zeros((BATCH, D_MODEL), jnp.float32)
    row_iota = lax.broadcasted_iota(jnp.int32, (CHUNK, BATCH), 0)
    e_col = (lax.broadcasted_iota(jnp.int32, (G * BATCH, G), 0) // CHUNK
             == lax.broadcasted_iota(jnp.int32, (G * BATCH, G), 1)
             ).astype(jnp.float32)
    e_mat = (lax.broadcasted_iota(jnp.int32, (DG, G), 0) // D_MODEL
             == lax.broadcasted_iota(jnp.int32, (DG, G), 1)
             ).astype(jnp.float32)

    a = jnp.concatenate(
        [cb_ref[pl.ds(t * CHUNK, CHUNK), :] for t in range(G)], axis=1)
    cn = lax.dot_general(a * a, e_mat, (((1,), (0,)), ((), ())),
                         precision=lax.Precision.HIGHEST,
                         preferred_element_type=jnp.float32)
    m2a = -2.0 * a
    m2ah = _hi(m2a)
    m2al = m2a - m2ah
    cnh = _hi(cn)
    cnl = cn - cnh
    a_ext = jnp.concatenate([m2ah, m2ah, m2al, cnh, cnl], axis=1)

    xh = _hi(x)
    xl = x - xh

    def blockdiag(v):
        return jnp.concatenate(
            [jnp.concatenate([v if t == u else zeros for u in range(G)], axis=1)
             for t in range(G)], axis=0)

    bh = blockdiag(xh)
    bl = blockdiag(xl)
    b_ext = jnp.concatenate([bh, bl, bh, e_col, e_col], axis=1)

    s_all = lax.dot_general(a_ext, b_ext, (((1,), (1,)), ((), ())),
                            preferred_element_type=jnp.float32)

    best_val = jnp.full((1, BATCH), jnp.inf, jnp.float32)
    best_idx = jnp.zeros((1, BATCH), jnp.int32)
    for t in range(G):
        s = s_all[:, t * BATCH:(t + 1) * BATCH]
        m = jnp.min(s, axis=0, keepdims=True)
        im = jnp.min(jnp.where(s == m, row_iota, jnp.int32(NUM_TOKENS)),
                     axis=0, keepdims=True) + t * CHUNK
        take = m < best_val
        best_val = jnp.where(take, m, best_val)
        best_idx = jnp.where(take, im, best_idx)
    idx_ref[...] = best_idx


def _gather_body(table_hbm, idx_hbm, out_hbm, idx_v, rows_v, sem):
    wid = lax.axis_index("s") * _NC + lax.axis_index("c")
    base = wid * _BPW
    pltpu.sync_copy(idx_hbm.at[pl.ds(base, _BPW)], idx_v)
    pltpu.async_copy(table_hbm.at[idx_v], rows_v, sem).wait()
    pltpu.sync_copy(rows_v, out_hbm.at[pl.ds(base, _BPW)])


def kernel(x, codebook, embed_table):
    argmin_call = pl.pallas_call(
        _argmin_body,
        out_shape=jax.ShapeDtypeStruct((1, BATCH), jnp.int32),
    )
    gather_call = pl.kernel(
        _gather_body,
        out_type=jax.ShapeDtypeStruct((BATCH, D_MODEL), jnp.float32),
        mesh=plsc.VectorSubcoreMesh(core_axis_name="c", subcore_axis_name="s"),
        scratch_types=[
            pltpu.VMEM((_BPW,), jnp.int32),
            pltpu.VMEM((_BPW, D_MODEL), jnp.float32),
            pltpu.SemaphoreType.DMA,
        ],
        compiler_params=pltpu.CompilerParams(use_tc_tiling_on_sc=False),
    )
    idx = argmin_call(x, codebook).reshape(BATCH)
    return gather_call(embed_table, idx)


# speedup vs baseline: 1.1103x; 1.1103x over previous
import jax
import jax.numpy as jnp
from jax import lax
from jax.experimental import pallas as pl
from jax.experimental.pallas import tpu as pltpu
from jax.experimental.pallas import tpu_sc as plsc

NUM_TOKENS = 8192
D_MODEL = 32
BATCH = 1024

CHUNK = 1024
G = NUM_TOKENS // CHUNK

_NC = 2
_NS = 16
_NW = _NC * _NS
_BPW = BATCH // _NW


def _hi(v):
    return v.astype(jnp.bfloat16).astype(jnp.float32)


def _argmin_body(x_ref, cb_ref, idx_ref):
    x = x_ref[...]
    cb = cb_ref[...]
    row_iota = lax.broadcasted_iota(jnp.int32, (CHUNK, BATCH), 0)
    ones_d1 = jnp.ones((D_MODEL, 1), jnp.float32)
    ones_b1 = jnp.ones((BATCH, 1), jnp.float32)

    cn = lax.dot_general(cb * cb, ones_d1, (((1,), (0,)), ((), ())),
                         precision=lax.Precision.HIGHEST,
                         preferred_element_type=jnp.float32)
    m2c = -2.0 * cb
    m2ch = _hi(m2c)
    m2cl = m2c - m2ch
    cnh = _hi(cn)
    cnl = cn - cnh
    cb_ext = jnp.concatenate([m2ch, m2ch, m2cl, cnh, cnl], axis=1)

    xh = _hi(x)
    xl = x - xh
    x_ext = jnp.concatenate([xh, xl, xh, ones_b1, ones_b1], axis=1)

    s_all = lax.dot_general(cb_ext, x_ext, (((1,), (1,)), ((), ())),
                            preferred_element_type=jnp.float32)

    best_val = jnp.full((1, BATCH), jnp.inf, jnp.float32)
    best_idx = jnp.zeros((1, BATCH), jnp.int32)
    for t in range(G):
        s = s_all[t * CHUNK:(t + 1) * CHUNK, :]
        m = jnp.min(s, axis=0, keepdims=True)
        im = jnp.min(jnp.where(s == m, row_iota, jnp.int32(NUM_TOKENS)),
                     axis=0, keepdims=True) + t * CHUNK
        take = m < best_val
        best_val = jnp.where(take, m, best_val)
        best_idx = jnp.where(take, im, best_idx)
    idx_ref[...] = best_idx


def _gather_body(table_hbm, idx_hbm, out_hbm, idx_v, rows_v, sem):
    wid = lax.axis_index("s") * _NC + lax.axis_index("c")
    base = wid * _BPW
    pltpu.sync_copy(idx_hbm.at[pl.ds(base, _BPW)], idx_v)
    pltpu.async_copy(table_hbm.at[idx_v], rows_v, sem).wait()
    pltpu.sync_copy(rows_v, out_hbm.at[pl.ds(base, _BPW)])


def kernel(x, codebook, embed_table):
    argmin_call = pl.pallas_call(
        _argmin_body,
        out_shape=jax.ShapeDtypeStruct((1, BATCH), jnp.int32),
    )
    gather_call = pl.kernel(
        _gather_body,
        out_type=jax.ShapeDtypeStruct((BATCH, D_MODEL), jnp.float32),
        mesh=plsc.VectorSubcoreMesh(core_axis_name="c", subcore_axis_name="s"),
        scratch_types=[
            pltpu.VMEM((_BPW,), jnp.int32),
            pltpu.VMEM((_BPW, D_MODEL), jnp.float32),
            pltpu.SemaphoreType.DMA,
        ],
        compiler_params=pltpu.CompilerParams(use_tc_tiling_on_sc=False),
    )
    idx = argmin_call(x, codebook).reshape(BATCH)
    return gather_call(embed_table, idx)
